# baseline (speedup 1.0000x reference)
import functools

import jax
import jax.numpy as jnp
from jax.experimental import pallas as pl
from jax.experimental.pallas import tpu as pltpu

ND = 4
D = 1024
B = 1024
S = 20
G = 16
PAD = B + ND * G
NBLK = PAD // G
BM = G * S


def _route(ids):
    onehot = (ids[:, None] == jnp.arange(ND, dtype=ids.dtype)[None, :]).astype(jnp.int32)
    ranks_all = jnp.cumsum(onehot, axis=0) - onehot
    rank = jnp.take_along_axis(ranks_all, ids[:, None], axis=1)[:, 0]
    counts = jnp.sum(onehot, axis=0)
    padded = ((counts + G - 1) // G) * G
    ends = jnp.cumsum(padded)
    starts = ends - padded
    dst = starts[ids] + rank
    sidx = jnp.zeros((PAD,), jnp.int32).at[dst].set(jnp.arange(B, dtype=jnp.int32))
    gs = jnp.arange(NBLK, dtype=jnp.int32) * G
    block_dom = jnp.minimum(jnp.sum((gs[:, None] >= ends[None, :]).astype(jnp.int32), axis=1), ND - 1)
    valid = jnp.clip(counts[block_dom] - (gs - starts[block_dom]), 0, G)
    return sidx, dst, counts, block_dom, valid


def _mm_body(bd_ref, vd_ref, x_ref, w_ref, a_ref, o_ref, l_ref):
    g = pl.program_id(0)
    x = x_ref[...]
    w = w_ref[0]
    res = jax.lax.dot_general(x, w, dimension_numbers=(((1,), (1,)), ((), ())),
                              preferred_element_type=jnp.float32)
    o_ref[...] = res
    nrows = vd_ref[g] * S
    rows = jax.lax.broadcasted_iota(jnp.int32, (BM, D), 0)
    diff = res - a_ref[...]
    sq = jnp.sum(jnp.where(rows < nrows, diff * diff, 0.0))
    lane = jax.lax.broadcasted_iota(jnp.int32, (1, 1, 128), 2)
    l_ref[...] = jnp.where(lane == 0, sq, 0.0)


def _project(xs2d, Ws, anchor_tiled, block_dom, valid, interpret=False):
    grid_spec = pltpu.PrefetchScalarGridSpec(
        num_scalar_prefetch=2,
        grid=(NBLK,),
        in_specs=[
            pl.BlockSpec((BM, D), lambda g, bd, vd: (g, 0)),
            pl.BlockSpec((1, D, D), lambda g, bd, vd: (bd[g], 0, 0)),
            pl.BlockSpec((BM, D), lambda g, bd, vd: (0, 0)),
        ],
        out_specs=[
            pl.BlockSpec((BM, D), lambda g, bd, vd: (g, 0)),
            pl.BlockSpec((1, 1, 128), lambda g, bd, vd: (g, 0, 0)),
        ],
    )
    return pl.pallas_call(
        _mm_body,
        grid_spec=grid_spec,
        out_shape=[
            jax.ShapeDtypeStruct((PAD * S, D), jnp.float32),
            jax.ShapeDtypeStruct((NBLK, 1, 128), jnp.float32),
        ],
        interpret=interpret,
    )(block_dom, valid, xs2d, Ws, anchor_tiled)


def kernel(features, domain_ids, anchor, Ws):
    ids = domain_ids.astype(jnp.int32)
    sidx, dst, counts, block_dom, valid = _route(ids)

    xs2d = features.reshape(B, S * D)[sidx].reshape(PAD * S, D)
    anchor_tiled = jnp.tile(anchor.reshape(S, D), (G, 1))

    out_sorted, loss_part = _project(xs2d, Ws, anchor_tiled, block_dom, valid)

    projected = out_sorted.reshape(PAD, S * D)[dst].reshape(B, S, D)

    sq_dom = jnp.zeros((ND,), jnp.float32).at[block_dom].add(loss_part[:, 0, 0])
    denom = (jnp.maximum(counts, 1) * S * D).astype(jnp.float32)
    loss = jnp.sum(jnp.where(counts > 0, sq_dom / denom, 0.0)) / ND
    return projected, loss


# speedup vs baseline: 1.0205x; 1.0205x over previous
import functools

import jax
import jax.numpy as jnp
from jax.experimental import pallas as pl
from jax.experimental.pallas import tpu as pltpu

ND = 4
D = 1024
B = 1024
S = 20
G = 16
PAD = B + ND * G
NBLK = PAD // G
BM = G * S


def _route(ids):
    onehot = (ids[:, None] == jnp.arange(ND, dtype=ids.dtype)[None, :]).astype(jnp.int32)
    ranks_all = jnp.cumsum(onehot, axis=0) - onehot
    rank = jnp.take_along_axis(ranks_all, ids[:, None], axis=1)[:, 0]
    counts = jnp.sum(onehot, axis=0)
    padded = ((counts + G - 1) // G) * G
    ends = jnp.cumsum(padded)
    starts = ends - padded
    dst = starts[ids] + rank
    sidx = jnp.zeros((PAD,), jnp.int32).at[dst].set(jnp.arange(B, dtype=jnp.int32))
    gs = jnp.arange(NBLK, dtype=jnp.int32) * G
    block_dom = jnp.minimum(jnp.sum((gs[:, None] >= ends[None, :]).astype(jnp.int32), axis=1), ND - 1)
    valid = jnp.clip(counts[block_dom] - (gs - starts[block_dom]), 0, G)
    return sidx, dst, counts, block_dom, valid


def _mm_body(bd_ref, vd_ref, x_ref, w_ref, a_ref, o_ref, l_ref):
    g = pl.program_id(0)
    x = x_ref[...]
    w = w_ref[0]
    res = jax.lax.dot_general(x, w, dimension_numbers=(((1,), (1,)), ((), ())),
                              preferred_element_type=jnp.float32)
    o_ref[...] = res
    nrows = vd_ref[g] * S
    rows = jax.lax.broadcasted_iota(jnp.int32, (BM, D), 0)
    diff = res - a_ref[...]
    sq = jnp.sum(jnp.where(rows < nrows, diff * diff, 0.0))
    lane = jax.lax.broadcasted_iota(jnp.int32, (1, 1, 128), 2)
    l_ref[...] = jnp.where(lane == 0, sq, 0.0)


def _project(xs2d, Ws, anchor_tiled, block_dom, valid, interpret=False):
    grid_spec = pltpu.PrefetchScalarGridSpec(
        num_scalar_prefetch=2,
        grid=(NBLK,),
        in_specs=[
            pl.BlockSpec((BM, D), lambda g, bd, vd: (g, 0)),
            pl.BlockSpec((1, D, D), lambda g, bd, vd: (bd[g], 0, 0)),
            pl.BlockSpec((BM, D), lambda g, bd, vd: (0, 0)),
        ],
        out_specs=[
            pl.BlockSpec((BM, D), lambda g, bd, vd: (g, 0)),
            pl.BlockSpec((1, 1, 128), lambda g, bd, vd: (g, 0, 0)),
        ],
    )
    return pl.pallas_call(
        _mm_body,
        grid_spec=grid_spec,
        out_shape=[
            jax.ShapeDtypeStruct((PAD * S, D), jnp.float32),
            jax.ShapeDtypeStruct((NBLK, 1, 128), jnp.float32),
        ],
        interpret=interpret,
    )(block_dom, valid, xs2d, Ws, anchor_tiled)


def kernel(features, domain_ids, anchor, Ws):
    ids = domain_ids.astype(jnp.int32)
    sidx, dst, counts, block_dom, valid = _route(ids)

    feats16 = features.astype(jnp.bfloat16)
    xs2d = feats16.reshape(B, S * D)[sidx].reshape(PAD * S, D)
    anchor_tiled = jnp.tile(anchor.reshape(S, D), (G, 1))

    out_sorted, loss_part = _project(xs2d, Ws.astype(jnp.bfloat16), anchor_tiled,
                                     block_dom, valid)

    projected = out_sorted.reshape(PAD, S * D)[dst].reshape(B, S, D)

    sq_dom = jnp.zeros((ND,), jnp.float32).at[block_dom].add(loss_part[:, 0, 0])
    denom = (jnp.maximum(counts, 1) * S * D).astype(jnp.float32)
    loss = jnp.sum(jnp.where(counts > 0, sq_dom / denom, 0.0)) / ND
    return projected, loss


# speedup vs baseline: 2.0052x; 1.9649x over previous
import jax
import jax.numpy as jnp
from jax.experimental import pallas as pl
from jax.experimental.pallas import tpu as pltpu

ND = 4
D = 1024
B = 1024
S = 20
G = 16
PAD = B + ND * G
NBLK = PAD // G
BM = G * S


def _route(ids):
    arange_b = jnp.arange(B, dtype=jnp.int32)
    onehot = (ids[:, None] == jnp.arange(ND, dtype=ids.dtype)[None, :]).astype(jnp.int32)
    ranks_all = jnp.cumsum(onehot, axis=0) - onehot
    rank = jnp.take_along_axis(ranks_all, ids[:, None], axis=1)[:, 0]
    counts = jnp.sum(onehot, axis=0)
    padded = ((counts + G - 1) // G) * G
    ends = jnp.cumsum(padded)
    starts = ends - padded
    dst = starts[ids] + rank
    firsttok = jnp.full((ND,), B, jnp.int32).at[ids].min(arange_b)
    firsttok = jnp.where(counts > 0, firsttok, 0)
    slot = jnp.arange(PAD, dtype=jnp.int32)
    slot_dom_raw = jnp.minimum(
        jnp.sum((slot[:, None] >= ends[None, :]).astype(jnp.int32), axis=1), ND - 1)
    in_real = slot < ends[ND - 1]
    slot_dom = jnp.where(in_real, slot_dom_raw, ids[0])
    fill = jnp.where(in_real, firsttok[slot_dom_raw], 0)
    src = fill.at[dst].set(arange_b)
    bd = slot_dom.reshape(NBLK, G)[:, 0]
    gs = jnp.arange(NBLK, dtype=jnp.int32) * G
    valid = jnp.clip(counts[bd] - (gs - starts[bd]), 0, G)
    return src, bd, valid, counts


def _fused_body(src_ref, bd_ref, vd_ref, feat_ref, w_ref, a_ref,
                out_ref, l_ref, xacc, racc, insem, outsem):
    g = pl.program_id(0)

    def issue_in(gg):
        pp = jax.lax.rem(gg, 2)
        for t in range(G):
            b = src_ref[gg * G + t]
            pltpu.make_async_copy(
                feat_ref.at[b], xacc.at[pp, t], insem.at[pp, t]
            ).start()

    def wait_in(gg):
        pp = jax.lax.rem(gg, 2)
        for t in range(G):
            pltpu.make_async_copy(
                feat_ref.at[0], xacc.at[pp, t], insem.at[pp, t]
            ).wait()

    def issue_out(gg):
        pp = jax.lax.rem(gg, 2)
        for t in range(G):
            b = src_ref[gg * G + t]
            pltpu.make_async_copy(
                racc.at[pp, t], out_ref.at[b], outsem.at[pp, t]
            ).start()

    def wait_out(gg):
        pp = jax.lax.rem(gg, 2)
        for t in range(G):
            pltpu.make_async_copy(
                racc.at[pp, t], out_ref.at[0], outsem.at[pp, t]
            ).wait()

    @pl.when(g == 0)
    def _():
        issue_in(jnp.int32(0))

    @pl.when(g + 1 < NBLK)
    def _():
        issue_in(g + 1)

    @pl.when(g >= 2)
    def _():
        wait_out(g - 2)

    @pl.when(g < NBLK)
    def _():
        p = jax.lax.rem(g, 2)
        wait_in(g)
        x = xacc[p].astype(jnp.bfloat16)
        w = w_ref[0]
        res = jax.lax.dot_general(x, w, dimension_numbers=(((2,), (1,)), ((), ())),
                                  preferred_element_type=jnp.float32)
        racc[p] = res
        toks = jax.lax.broadcasted_iota(jnp.int32, (G, S, D), 0)
        diff = res - a_ref[...]
        sq = jnp.sum(jnp.where(toks < vd_ref[g], diff * diff, 0.0))
        lane = jax.lax.broadcasted_iota(jnp.int32, (1, 1, 128), 2)
        l_ref[...] = jnp.where(lane == 0, sq, 0.0)
        issue_out(g)


def _run_fused(features, Ws16, anchor_tiled, src, bd, valid, interpret=False):
    grid_spec = pltpu.PrefetchScalarGridSpec(
        num_scalar_prefetch=3,
        grid=(NBLK + 2,),
        in_specs=[
            pl.BlockSpec(memory_space=pl.ANY),
            pl.BlockSpec((1, D, D), lambda g, src, bd, vd: (bd[jnp.minimum(g, NBLK - 1)], 0, 0)),
            pl.BlockSpec((G, S, D), lambda g, src, bd, vd: (0, 0, 0)),
        ],
        out_specs=[
            pl.BlockSpec(memory_space=pl.ANY),
            pl.BlockSpec((1, 1, 128), lambda g, src, bd, vd: (jnp.minimum(g, NBLK - 1), 0, 0)),
        ],
        scratch_shapes=[
            pltpu.VMEM((2, G, S, D), jnp.float32),
            pltpu.VMEM((2, G, S, D), jnp.float32),
            pltpu.SemaphoreType.DMA((2, G)),
            pltpu.SemaphoreType.DMA((2, G)),
        ],
    )
    return pl.pallas_call(
        _fused_body,
        grid_spec=grid_spec,
        out_shape=[
            jax.ShapeDtypeStruct((B, S, D), jnp.float32),
            jax.ShapeDtypeStruct((NBLK, 1, 128), jnp.float32),
        ],
        interpret=interpret,
    )(src, bd, valid, features, Ws16, anchor_tiled)


def kernel(features, domain_ids, anchor, Ws):
    ids = domain_ids.astype(jnp.int32)
    src, bd, valid, counts = _route(ids)
    anchor_tiled = jnp.broadcast_to(anchor.reshape(1, S, D), (G, S, D))

    projected, loss_part = _run_fused(
        features, Ws.astype(jnp.bfloat16), anchor_tiled, src, bd, valid)

    sq_dom = jnp.zeros((ND,), jnp.float32).at[bd].add(loss_part[:, 0, 0])
    denom = (jnp.maximum(counts, 1) * S * D).astype(jnp.float32)
    loss = jnp.sum(jnp.where(counts > 0, sq_dom / denom, 0.0)) / ND
    return projected, loss


# speedup vs baseline: 3.6227x; 1.8067x over previous
import jax
import jax.numpy as jnp
from jax.experimental import pallas as pl
from jax.experimental.pallas import tpu as pltpu

ND = 4
D = 1024
B = 1024
S = 20
G = 16
PAD = B + ND * G
NBLK = PAD // G


def _route(ids):
    order = jnp.argsort(ids, stable=True).astype(jnp.int32)
    onehot = (ids[:, None] == jnp.arange(ND, dtype=ids.dtype)[None, :]).astype(jnp.int32)
    counts = jnp.sum(onehot, axis=0)
    real_ends = jnp.cumsum(counts)
    real_starts = real_ends - counts
    padded = ((counts + G - 1) // G) * G
    ends = jnp.cumsum(padded)
    starts = ends - padded

    slot = jnp.arange(PAD, dtype=jnp.int32)
    sdr = jnp.minimum(
        jnp.sum((slot[:, None] >= ends[None, :]).astype(jnp.int32), axis=1), ND - 1)
    in_real = slot < ends[ND - 1]
    q = slot - starts[sdr]
    pos = jnp.where(in_real & (q < counts[sdr]), real_starts[sdr] + q,
                    jnp.where(in_real, real_starts[sdr], 0))
    src = order[pos]

    d0 = ids[order[0]].astype(jnp.int32)
    bd = jnp.where(in_real, sdr, d0).reshape(NBLK, G)[:, 0]
    gs = jnp.arange(NBLK, dtype=jnp.int32) * G
    valid = jnp.clip(counts[bd] - (gs - starts[bd]), 0, G)
    return src, bd, valid, counts


def _fused_body(src_ref, bd_ref, vd_ref, feat_ref, w_ref, a_ref,
                out_ref, l_ref, xacc, racc, insem, outsem):
    g = pl.program_id(0)

    def issue_in(gg):
        pp = jax.lax.rem(gg, 2)
        for t in range(G):
            b = src_ref[gg * G + t]
            pltpu.make_async_copy(
                feat_ref.at[:, b, :], xacc.at[pp, t], insem.at[pp, t]
            ).start()

    def wait_in(gg):
        pp = jax.lax.rem(gg, 2)
        for t in range(G):
            pltpu.make_async_copy(
                feat_ref.at[:, 0, :], xacc.at[pp, t], insem.at[pp, t]
            ).wait()

    def issue_out(gg):
        pp = jax.lax.rem(gg, 2)
        for t in range(G):
            b = src_ref[gg * G + t]
            pltpu.make_async_copy(
                racc.at[pp, t], out_ref.at[:, b, :], outsem.at[pp, t]
            ).start()

    def wait_out(gg):
        pp = jax.lax.rem(gg, 2)
        for t in range(G):
            pltpu.make_async_copy(
                racc.at[pp, t], out_ref.at[:, 0, :], outsem.at[pp, t]
            ).wait()

    @pl.when(g == 0)
    def _():
        issue_in(jnp.int32(0))

    @pl.when(g + 1 < NBLK)
    def _():
        issue_in(g + 1)

    @pl.when(g >= 2)
    def _():
        wait_out(g - 2)

    @pl.when(g < NBLK)
    def _():
        p = jax.lax.rem(g, 2)
        wait_in(g)
        x = xacc[p].astype(jnp.bfloat16)
        w = w_ref[0]
        res = jax.lax.dot_general(x, w, dimension_numbers=(((2,), (1,)), ((), ())),
                                  preferred_element_type=jnp.float32)
        racc[p] = res
        toks = jax.lax.broadcasted_iota(jnp.int32, (G, S, D), 0)
        diff = res - a_ref[...]
        sq = jnp.sum(jnp.where(toks < vd_ref[g], diff * diff, 0.0))
        lane = jax.lax.broadcasted_iota(jnp.int32, (1, 1, 128), 2)
        l_ref[...] = jnp.where(lane == 0, sq, 0.0)
        issue_out(g)


def _run_fused(feats_t, Ws16, anchor_tiled, src, bd, valid, interpret=False):
    grid_spec = pltpu.PrefetchScalarGridSpec(
        num_scalar_prefetch=3,
        grid=(NBLK + 2,),
        in_specs=[
            pl.BlockSpec(memory_space=pl.ANY),
            pl.BlockSpec((1, D, D), lambda g, src, bd, vd: (bd[jnp.minimum(g, NBLK - 1)], 0, 0)),
            pl.BlockSpec((G, S, D), lambda g, src, bd, vd: (0, 0, 0)),
        ],
        out_specs=[
            pl.BlockSpec(memory_space=pl.ANY),
            pl.BlockSpec((1, 1, 128), lambda g, src, bd, vd: (jnp.minimum(g, NBLK - 1), 0, 0)),
        ],
        scratch_shapes=[
            pltpu.VMEM((2, G, S, D), jnp.float32),
            pltpu.VMEM((2, G, S, D), jnp.float32),
            pltpu.SemaphoreType.DMA((2, G)),
            pltpu.SemaphoreType.DMA((2, G)),
        ],
    )
    return pl.pallas_call(
        _fused_body,
        grid_spec=grid_spec,
        out_shape=[
            jax.ShapeDtypeStruct((S, B, D), jnp.float32),
            jax.ShapeDtypeStruct((NBLK, 1, 128), jnp.float32),
        ],
        interpret=interpret,
    )(src, bd, valid, feats_t, Ws16, anchor_tiled)


def kernel(features, domain_ids, anchor, Ws):
    ids = domain_ids.astype(jnp.int32)
    src, bd, valid, counts = _route(ids)
    anchor_tiled = jnp.broadcast_to(anchor.reshape(1, S, D), (G, S, D))
    feats_t = jnp.transpose(features, (1, 0, 2))

    out_t, loss_part = _run_fused(
        feats_t, Ws.astype(jnp.bfloat16), anchor_tiled, src, bd, valid)
    projected = jnp.transpose(out_t, (1, 0, 2))

    bd_onehot = (bd[:, None] == jnp.arange(ND, dtype=jnp.int32)[None, :]).astype(jnp.float32)
    sq_dom = jnp.sum(loss_part[:, 0, 0][:, None] * bd_onehot, axis=0)
    denom = (jnp.maximum(counts, 1) * S * D).astype(jnp.float32)
    loss = jnp.sum(jnp.where(counts > 0, sq_dom / denom, 0.0)) / ND
    return projected, loss


# speedup vs baseline: 4.0283x; 1.1120x over previous
import jax
import jax.numpy as jnp
from jax.experimental import pallas as pl
from jax.experimental.pallas import tpu as pltpu

ND = 4
D = 1024
B = 1024
S = 20
G = 16
PAD = B + ND * G
NBLK = PAD // G


def _route(ids):
    order = jnp.argsort(ids, stable=True).astype(jnp.int32)
    onehot = (ids[:, None] == jnp.arange(ND, dtype=ids.dtype)[None, :]).astype(jnp.int32)
    counts = jnp.sum(onehot, axis=0)
    rstarts = jnp.cumsum(counts) - counts
    padded = ((counts + G - 1) // G) * G
    ends = jnp.cumsum(padded)
    starts = ends - padded

    gs = jnp.arange(NBLK, dtype=jnp.int32) * G
    bdr = jnp.minimum(
        jnp.sum((gs[:, None] >= ends[None, :]).astype(jnp.int32), axis=1), ND - 1)
    d0 = ids[order[0]].astype(jnp.int32)
    bd = jnp.where(gs < ends[ND - 1], bdr, d0)
    return order, starts, rstarts, counts, bd


def _fused_body(order_ref, st_ref, rst_ref, cnt_ref, bd_ref,
                feat_ref, w_ref, a_ref, out_ref, l_ref,
                xacc, racc, wb, insem, outsem):
    g = pl.program_id(0)

    def slot_src(gg, t):
        bdv = bd_ref[gg]
        q = gg * G + t - st_ref[bdv]
        qq = jnp.where(q < cnt_ref[bdv], q, 0)
        return order_ref[rst_ref[bdv] + qq]

    def issue_in(gg):
        pp = jax.lax.rem(gg, 2)
        for t in range(G):
            b = slot_src(gg, t)
            pltpu.make_async_copy(
                feat_ref.at[:, b, :], xacc.at[pp, t], insem.at[pp, t]
            ).start()

    def wait_in(pp):
        for t in range(G):
            pltpu.make_async_copy(
                feat_ref.at[:, 0, :], xacc.at[pp, t], insem.at[pp, t]
            ).wait()

    def issue_out(gg):
        pp = jax.lax.rem(gg, 2)
        for t in range(G):
            b = slot_src(gg, t)
            pltpu.make_async_copy(
                racc.at[pp, t], out_ref.at[:, b, :], outsem.at[pp, t]
            ).start()

    def wait_out(pp):
        for t in range(G):
            pltpu.make_async_copy(
                racc.at[pp, t], out_ref.at[:, 0, :], outsem.at[pp, t]
            ).wait()

    @pl.when(g == 0)
    def _():
        issue_in(jnp.int32(0))

    @pl.when(g + 1 < NBLK)
    def _():
        issue_in(g + 1)

    @pl.when(g >= 2)
    def _():
        wait_out(jax.lax.rem(g, 2))

    @pl.when(g < NBLK)
    def _():
        p = jax.lax.rem(g, 2)
        wait_in(p)

        @pl.when((g == 0) | (bd_ref[jnp.maximum(g - 1, 0)] != bd_ref[g]))
        def _():
            wb[...] = w_ref[0].astype(jnp.bfloat16)

        x = xacc[p].astype(jnp.bfloat16)
        w = wb[...]
        res = jax.lax.dot_general(x, w, dimension_numbers=(((2,), (1,)), ((), ())),
                                  preferred_element_type=jnp.float32)
        racc[p] = res
        bdv = bd_ref[g]
        nvalid = jnp.clip(cnt_ref[bdv] - (g * G - st_ref[bdv]), 0, G)
        toks = jax.lax.broadcasted_iota(jnp.int32, (G, S, D), 0)
        diff = res - a_ref[...]
        sq = jnp.sum(jnp.where(toks < nvalid, diff * diff, 0.0))
        lane = jax.lax.broadcasted_iota(jnp.int32, (1, 1, 128), 2)
        l_ref[...] = jnp.where(lane == 0, sq, 0.0)
        issue_out(g)


def _run_fused(feats_t, Ws, anchor_tiled, order, starts, rstarts, counts, bd,
               interpret=False):
    grid_spec = pltpu.PrefetchScalarGridSpec(
        num_scalar_prefetch=5,
        grid=(NBLK + 2,),
        in_specs=[
            pl.BlockSpec(memory_space=pl.ANY),
            pl.BlockSpec((1, D, D),
                         lambda g, o, st, rst, cnt, bd: (bd[jnp.minimum(g, NBLK - 1)], 0, 0)),
            pl.BlockSpec((G, S, D), lambda g, o, st, rst, cnt, bd: (0, 0, 0)),
        ],
        out_specs=[
            pl.BlockSpec(memory_space=pl.ANY),
            pl.BlockSpec((1, 1, 128),
                         lambda g, o, st, rst, cnt, bd: (jnp.minimum(g, NBLK - 1), 0, 0)),
        ],
        scratch_shapes=[
            pltpu.VMEM((2, G, S, D), jnp.float32),
            pltpu.VMEM((2, G, S, D), jnp.float32),
            pltpu.VMEM((D, D), jnp.bfloat16),
            pltpu.SemaphoreType.DMA((2, G)),
            pltpu.SemaphoreType.DMA((2, G)),
        ],
    )
    return pl.pallas_call(
        _fused_body,
        grid_spec=grid_spec,
        out_shape=[
            jax.ShapeDtypeStruct((S, B, D), jnp.float32),
            jax.ShapeDtypeStruct((NBLK, 1, 128), jnp.float32),
        ],
        interpret=interpret,
    )(order, starts, rstarts, counts, bd, feats_t, Ws, anchor_tiled)


def kernel(features, domain_ids, anchor, Ws):
    ids = domain_ids.astype(jnp.int32)
    order, starts, rstarts, counts, bd = _route(ids)
    anchor_tiled = jnp.broadcast_to(anchor.reshape(1, S, D), (G, S, D))
    feats_t = jnp.transpose(features, (1, 0, 2))

    out_t, loss_part = _run_fused(
        feats_t, Ws, anchor_tiled, order, starts, rstarts, counts, bd)
    projected = jnp.transpose(out_t, (1, 0, 2))

    bd_onehot = (bd[:, None] == jnp.arange(ND, dtype=jnp.int32)[None, :]).astype(jnp.float32)
    sq_dom = jnp.sum(loss_part[:, 0, 0][:, None] * bd_onehot, axis=0)
    denom = (jnp.maximum(counts, 1) * S * D).astype(jnp.float32)
    loss = jnp.sum(jnp.where(counts > 0, sq_dom / denom, 0.0)) / ND
    return projected, loss


# speedup vs baseline: 4.1447x; 1.0289x over previous
import jax
import jax.numpy as jnp
from jax.experimental import pallas as pl
from jax.experimental.pallas import tpu as pltpu

ND = 4
D = 1024
B = 1024
S = 20
G = 32
PAD = B + ND * G
NBLK = PAD // G


def _route(ids):
    order = jnp.argsort(ids, stable=True).astype(jnp.int32)
    onehot = (ids[:, None] == jnp.arange(ND, dtype=ids.dtype)[None, :]).astype(jnp.int32)
    counts = jnp.sum(onehot, axis=0)
    rstarts = jnp.cumsum(counts) - counts
    padded = ((counts + G - 1) // G) * G
    ends = jnp.cumsum(padded)
    starts = ends - padded

    gs = jnp.arange(NBLK, dtype=jnp.int32) * G
    bdr = jnp.minimum(
        jnp.sum((gs[:, None] >= ends[None, :]).astype(jnp.int32), axis=1), ND - 1)
    d0 = ids[order[0]].astype(jnp.int32)
    bd = jnp.where(gs < ends[ND - 1], bdr, d0)
    return order, starts, rstarts, counts, bd


def _fused_body(order_ref, st_ref, rst_ref, cnt_ref, bd_ref,
                feat_ref, w_ref, a_ref, out_ref, l_ref,
                xacc, racc, wb, insem, outsem):
    g = pl.program_id(0)

    def slot_src(gg, t):
        bdv = bd_ref[gg]
        q = gg * G + t - st_ref[bdv]
        qq = jnp.where(q < cnt_ref[bdv], q, 0)
        return order_ref[rst_ref[bdv] + qq]

    def issue_in(gg):
        pp = jax.lax.rem(gg, 2)
        for t in range(G):
            b = slot_src(gg, t)
            pltpu.make_async_copy(
                feat_ref.at[:, b, :], xacc.at[pp, t], insem.at[pp, t]
            ).start()

    def wait_in(pp):
        for t in range(G):
            pltpu.make_async_copy(
                feat_ref.at[:, 0, :], xacc.at[pp, t], insem.at[pp, t]
            ).wait()

    def issue_out(gg):
        pp = jax.lax.rem(gg, 2)
        for t in range(G):
            b = slot_src(gg, t)
            pltpu.make_async_copy(
                racc.at[pp, t], out_ref.at[:, b, :], outsem.at[pp, t]
            ).start()

    def wait_out(pp):
        for t in range(G):
            pltpu.make_async_copy(
                racc.at[pp, t], out_ref.at[:, 0, :], outsem.at[pp, t]
            ).wait()

    @pl.when(g == 0)
    def _():
        issue_in(jnp.int32(0))

    @pl.when(g + 1 < NBLK)
    def _():
        issue_in(g + 1)

    @pl.when(g >= 2)
    def _():
        wait_out(jax.lax.rem(g, 2))

    @pl.when(g < NBLK)
    def _():
        p = jax.lax.rem(g, 2)
        wait_in(p)

        @pl.when((g == 0) | (bd_ref[jnp.maximum(g - 1, 0)] != bd_ref[g]))
        def _():
            wb[...] = w_ref[0].astype(jnp.bfloat16)

        x = xacc[p].astype(jnp.bfloat16)
        w = wb[...]
        res = jax.lax.dot_general(x, w, dimension_numbers=(((2,), (1,)), ((), ())),
                                  preferred_element_type=jnp.float32)
        racc[p] = res
        bdv = bd_ref[g]
        nvalid = jnp.clip(cnt_ref[bdv] - (g * G - st_ref[bdv]), 0, G)
        toks = jax.lax.broadcasted_iota(jnp.int32, (G, S, D), 0)
        diff = res - a_ref[...]
        sq = jnp.sum(jnp.where(toks < nvalid, diff * diff, 0.0))
        lane = jax.lax.broadcasted_iota(jnp.int32, (1, 1, 128), 2)
        l_ref[...] = jnp.where(lane == 0, sq, 0.0)
        issue_out(g)


def _run_fused(feats_t, Ws, anchor_tiled, order, starts, rstarts, counts, bd,
               interpret=False):
    grid_spec = pltpu.PrefetchScalarGridSpec(
        num_scalar_prefetch=5,
        grid=(NBLK + 2,),
        in_specs=[
            pl.BlockSpec(memory_space=pl.ANY),
            pl.BlockSpec((1, D, D),
                         lambda g, o, st, rst, cnt, bd: (bd[jnp.minimum(g, NBLK - 1)], 0, 0)),
            pl.BlockSpec((G, S, D), lambda g, o, st, rst, cnt, bd: (0, 0, 0)),
        ],
        out_specs=[
            pl.BlockSpec(memory_space=pl.ANY),
            pl.BlockSpec((1, 1, 128),
                         lambda g, o, st, rst, cnt, bd: (jnp.minimum(g, NBLK - 1), 0, 0)),
        ],
        scratch_shapes=[
            pltpu.VMEM((2, G, S, D), jnp.float32),
            pltpu.VMEM((2, G, S, D), jnp.float32),
            pltpu.VMEM((D, D), jnp.bfloat16),
            pltpu.SemaphoreType.DMA((2, G)),
            pltpu.SemaphoreType.DMA((2, G)),
        ],
    )
    return pl.pallas_call(
        _fused_body,
        grid_spec=grid_spec,
        out_shape=[
            jax.ShapeDtypeStruct((S, B, D), jnp.float32),
            jax.ShapeDtypeStruct((NBLK, 1, 128), jnp.float32),
        ],
        interpret=interpret,
    )(order, starts, rstarts, counts, bd, feats_t, Ws, anchor_tiled)


def kernel(features, domain_ids, anchor, Ws):
    ids = domain_ids.astype(jnp.int32)
    order, starts, rstarts, counts, bd = _route(ids)
    anchor_tiled = jnp.broadcast_to(anchor.reshape(1, S, D), (G, S, D))
    feats_t = jnp.transpose(features, (1, 0, 2))

    out_t, loss_part = _run_fused(
        feats_t, Ws, anchor_tiled, order, starts, rstarts, counts, bd)
    projected = jnp.transpose(out_t, (1, 0, 2))

    bd_onehot = (bd[:, None] == jnp.arange(ND, dtype=jnp.int32)[None, :]).astype(jnp.float32)
    sq_dom = jnp.sum(loss_part[:, 0, 0][:, None] * bd_onehot, axis=0)
    denom = (jnp.maximum(counts, 1) * S * D).astype(jnp.float32)
    loss = jnp.sum(jnp.where(counts > 0, sq_dom / denom, 0.0)) / ND
    return projected, loss


# speedup vs baseline: 4.4442x; 1.0722x over previous
import jax
import jax.numpy as jnp
from jax.experimental import pallas as pl
from jax.experimental.pallas import tpu as pltpu

ND = 4
D = 1024
B = 1024
S = 20
G = 32
PAD = B + ND * G
NBLK = PAD // G


def _route(ids):
    order = jnp.argsort(ids, stable=True).astype(jnp.int32)
    onehot = (ids[:, None] == jnp.arange(ND, dtype=ids.dtype)[None, :]).astype(jnp.int32)
    counts = jnp.sum(onehot, axis=0)
    rstarts = jnp.cumsum(counts) - counts
    padded = ((counts + G - 1) // G) * G
    ends = jnp.cumsum(padded)
    starts = ends - padded

    gs = jnp.arange(NBLK, dtype=jnp.int32) * G
    bdr = jnp.minimum(
        jnp.sum((gs[:, None] >= ends[None, :]).astype(jnp.int32), axis=1), ND - 1)
    d0 = ids[order[0]].astype(jnp.int32)
    bd = jnp.where(gs < ends[ND - 1], bdr, d0)
    return order, starts, rstarts, counts, bd


def _fused_body(order_ref, st_ref, rst_ref, cnt_ref, bd_ref,
                feat_ref, w_ref, a_ref, out_ref, l_ref,
                xacc, racc, wb, bsave, insem, outsem):
    g = pl.program_id(0)

    def issue_in(gg):
        pp = jax.lax.rem(gg, 2)
        bdv = bd_ref[gg]
        base = gg * G - st_ref[bdv]
        cntv = cnt_ref[bdv]
        rstv = rst_ref[bdv]
        for t in range(G):
            q = base + t
            b = order_ref[rstv + jnp.where(q < cntv, q, 0)]
            bsave[pp, t] = b
            pltpu.make_async_copy(
                feat_ref.at[:, b, :], xacc.at[pp, t], insem.at[pp, t]
            ).start()

    def wait_in(pp):
        for t in range(G):
            pltpu.make_async_copy(
                feat_ref.at[:, 0, :], xacc.at[pp, t], insem.at[pp, t]
            ).wait()

    def issue_out(gg):
        pp = jax.lax.rem(gg, 2)
        for t in range(G):
            b = bsave[pp, t]
            pltpu.make_async_copy(
                racc.at[pp, t], out_ref.at[:, b, :], outsem.at[pp, t]
            ).start()

    def wait_out(pp):
        for t in range(G):
            pltpu.make_async_copy(
                racc.at[pp, t], out_ref.at[:, 0, :], outsem.at[pp, t]
            ).wait()

    @pl.when(g == 0)
    def _():
        issue_in(jnp.int32(0))

    @pl.when(g + 1 < NBLK)
    def _():
        issue_in(g + 1)

    @pl.when(g >= 2)
    def _():
        wait_out(jax.lax.rem(g, 2))

    @pl.when(g < NBLK)
    def _():
        p = jax.lax.rem(g, 2)
        wait_in(p)

        @pl.when((g == 0) | (bd_ref[jnp.maximum(g - 1, 0)] != bd_ref[g]))
        def _():
            wb[...] = w_ref[0].astype(jnp.bfloat16)

        x = xacc[p].astype(jnp.bfloat16)
        w = wb[...]
        res = jax.lax.dot_general(x, w, dimension_numbers=(((2,), (1,)), ((), ())),
                                  preferred_element_type=jnp.float32)
        racc[p] = res
        bdv = bd_ref[g]
        nvalid = jnp.clip(cnt_ref[bdv] - (g * G - st_ref[bdv]), 0, G)
        diff = res - a_ref[...]
        lane = jax.lax.broadcasted_iota(jnp.int32, (1, 1, 128), 2)

        @pl.when(nvalid == G)
        def _():
            sq = jnp.sum(diff * diff)
            l_ref[...] = jnp.where(lane == 0, sq, 0.0)

        @pl.when(nvalid < G)
        def _():
            toks = jax.lax.broadcasted_iota(jnp.int32, (G, S, D), 0)
            sq = jnp.sum(jnp.where(toks < nvalid, diff * diff, 0.0))
            l_ref[...] = jnp.where(lane == 0, sq, 0.0)

        issue_out(g)


def _run_fused(feats_t, Ws, anchor_tiled, order, starts, rstarts, counts, bd,
               interpret=False):
    grid_spec = pltpu.PrefetchScalarGridSpec(
        num_scalar_prefetch=5,
        grid=(NBLK + 2,),
        in_specs=[
            pl.BlockSpec(memory_space=pl.ANY),
            pl.BlockSpec((1, D, D),
                         lambda g, o, st, rst, cnt, bd: (bd[jnp.minimum(g, NBLK - 1)], 0, 0)),
            pl.BlockSpec((G, S, D), lambda g, o, st, rst, cnt, bd: (0, 0, 0)),
        ],
        out_specs=[
            pl.BlockSpec(memory_space=pl.ANY),
            pl.BlockSpec((1, 1, 128),
                         lambda g, o, st, rst, cnt, bd: (jnp.minimum(g, NBLK - 1), 0, 0)),
        ],
        scratch_shapes=[
            pltpu.VMEM((2, G, S, D), jnp.float32),
            pltpu.VMEM((2, G, S, D), jnp.float32),
            pltpu.VMEM((D, D), jnp.bfloat16),
            pltpu.SMEM((2, G), jnp.int32),
            pltpu.SemaphoreType.DMA((2, G)),
            pltpu.SemaphoreType.DMA((2, G)),
        ],
    )
    return pl.pallas_call(
        _fused_body,
        grid_spec=grid_spec,
        out_shape=[
            jax.ShapeDtypeStruct((S, B, D), jnp.float32),
            jax.ShapeDtypeStruct((NBLK, 1, 128), jnp.float32),
        ],
        interpret=interpret,
    )(order, starts, rstarts, counts, bd, feats_t, Ws, anchor_tiled)


def kernel(features, domain_ids, anchor, Ws):
    ids = domain_ids.astype(jnp.int32)
    order, starts, rstarts, counts, bd = _route(ids)
    anchor_tiled = jnp.broadcast_to(anchor.reshape(1, S, D), (G, S, D))
    feats_t = jnp.transpose(features, (1, 0, 2))

    out_t, loss_part = _run_fused(
        feats_t, Ws, anchor_tiled, order, starts, rstarts, counts, bd)
    projected = jnp.transpose(out_t, (1, 0, 2))

    bd_onehot = (bd[:, None] == jnp.arange(ND, dtype=jnp.int32)[None, :]).astype(jnp.float32)
    sq_dom = jnp.sum(loss_part[:, 0, 0][:, None] * bd_onehot, axis=0)
    denom = (jnp.maximum(counts, 1) * S * D).astype(jnp.float32)
    loss = jnp.sum(jnp.where(counts > 0, sq_dom / denom, 0.0)) / ND
    return projected, loss
